# Initial kernel scaffold; baseline (speedup 1.0000x reference)
import jax
import jax.numpy as jnp
from jax.experimental import pallas as pl


def kernel(x, W1, b1, Wconv, bconv, W2, b2):
    raise NotImplementedError("write your pallas kernel here")


# speedup vs baseline: 16.2287x; 16.2287x over previous
import jax
import jax.numpy as jnp
from jax.experimental import pallas as pl

_B, _N, _DIN, _E = 16, 1024, 2, 128
_K = 10
_R = 256


def _body(xr_ref, xt_ref, xf_ref, w1_ref, wcv_ref, w2_ref, b1p_ref, bcv_ref,
          out_ref):
    xr = xr_ref[0]
    xt = xt_ref[0]
    xf = xf_ref[0]

    x0c = xt[0:1, :]
    x1c = xt[1:2, :]
    sq_c = x0c * x0c + x1c * x1c
    x0r = xr[:, 0:1]
    x1r = xr[:, 1:2]
    sq_r = x0r * x0r + x1r * x1r
    dot = jax.lax.dot(xr, xt, preferred_element_type=jnp.float32)
    d2 = (sq_r + sq_c) - 2.0 * dot
    vals = jnp.maximum(d2, 1e-12)

    iota = jax.lax.broadcasted_iota(jnp.int32, (_R, _N), 1)
    cols = []
    for _ in range(_K):
        vmin = jnp.min(vals, axis=1, keepdims=True)
        tie = vals == vmin
        sel = jnp.max(jnp.where(tie, iota, -1), axis=1, keepdims=True)
        onehot = iota == sel
        vals = jnp.where(onehot, jnp.float32(jnp.inf), vals)
        cols.append(jax.lax.dot(onehot.astype(jnp.float32), xf))
    win = jnp.concatenate(cols, axis=1)

    wfold = jax.lax.dot(wcv_ref[...], w2_ref[...])
    bias = b1p_ref[...] + jax.lax.dot(bcv_ref[...], w2_ref[...])
    out_ref[0] = jax.lax.dot(xr, w1_ref[...]) + jax.lax.dot(win, wfold) + bias


def kernel(x, W1, b1, Wconv, bconv, W2, b2):
    xt = jnp.transpose(x, (0, 2, 1))
    wcv = jnp.transpose(Wconv, (2, 1, 0))[::-1].reshape(2 * _K, _E)
    b1p = (b1 + b2).reshape(1, _E)
    bcv = bconv.reshape(1, _E)

    grid = (_B, _N // _R)
    return pl.pallas_call(
        _body,
        grid=grid,
        in_specs=[
            pl.BlockSpec((1, _R, _DIN), lambda b, r: (b, r, 0)),
            pl.BlockSpec((1, _DIN, _N), lambda b, r: (b, 0, 0)),
            pl.BlockSpec((1, _N, _DIN), lambda b, r: (b, 0, 0)),
            pl.BlockSpec((_DIN, _E), lambda b, r: (0, 0)),
            pl.BlockSpec((2 * _K, _E), lambda b, r: (0, 0)),
            pl.BlockSpec((_E, _E), lambda b, r: (0, 0)),
            pl.BlockSpec((1, _E), lambda b, r: (0, 0)),
            pl.BlockSpec((1, _E), lambda b, r: (0, 0)),
        ],
        out_specs=pl.BlockSpec((1, _R, _E), lambda b, r: (b, r, 0)),
        out_shape=jax.ShapeDtypeStruct((_B, _N, _E), jnp.float32),
    )(x, xt, x, W1, wcv, W2, b1p, bcv)


# speedup vs baseline: 21.9220x; 1.3508x over previous
import jax
import jax.numpy as jnp
from jax.experimental import pallas as pl

_B, _N, _DIN, _E = 16, 1024, 2, 128
_K = 10
_R = 512


def _body(xr_ref, xt_ref, xf_ref, w1_ref, wcv_ref, w2_ref, b1p_ref, bcv_ref,
          out_ref):
    xr = xr_ref[0]
    xt = xt_ref[0]
    xf = xf_ref[0]

    x0c = xt[0:1, :]
    x1c = xt[1:2, :]
    sq_c = x0c * x0c + x1c * x1c
    x0r = xr[:, 0:1]
    x1r = xr[:, 1:2]
    sq_r = x0r * x0r + x1r * x1r
    dot = jax.lax.dot(xr, xt, preferred_element_type=jnp.float32)
    d2 = (sq_r + sq_c) - 2.0 * dot
    vals = jnp.maximum(d2, 1e-12)

    iotaf = jax.lax.broadcasted_iota(jnp.int32, (_R, _N), 1).astype(jnp.float32)
    cols = []
    for _ in range(_K):
        vmin = jnp.min(vals, axis=1, keepdims=True)
        tie = vals == vmin
        sel = jnp.max(jnp.where(tie, iotaf, -1.0), axis=1, keepdims=True)
        onehot = iotaf == sel
        vals = jnp.where(onehot, jnp.float32(jnp.inf), vals)
        cols.append(jax.lax.dot(onehot.astype(jnp.float32), xf))
    win = jnp.concatenate(cols, axis=1)

    wfold = jax.lax.dot(wcv_ref[...], w2_ref[...])
    bias = b1p_ref[...] + jax.lax.dot(bcv_ref[...], w2_ref[...])
    out_ref[0] = jax.lax.dot(xr, w1_ref[...]) + jax.lax.dot(win, wfold) + bias


def kernel(x, W1, b1, Wconv, bconv, W2, b2):
    xt = jnp.transpose(x, (0, 2, 1))
    wcv = jnp.transpose(Wconv, (2, 1, 0))[::-1].reshape(2 * _K, _E)
    b1p = (b1 + b2).reshape(1, _E)
    bcv = bconv.reshape(1, _E)

    grid = (_B, _N // _R)
    return pl.pallas_call(
        _body,
        grid=grid,
        in_specs=[
            pl.BlockSpec((1, _R, _DIN), lambda b, r: (b, r, 0)),
            pl.BlockSpec((1, _DIN, _N), lambda b, r: (b, 0, 0)),
            pl.BlockSpec((1, _N, _DIN), lambda b, r: (b, 0, 0)),
            pl.BlockSpec((_DIN, _E), lambda b, r: (0, 0)),
            pl.BlockSpec((2 * _K, _E), lambda b, r: (0, 0)),
            pl.BlockSpec((_E, _E), lambda b, r: (0, 0)),
            pl.BlockSpec((1, _E), lambda b, r: (0, 0)),
            pl.BlockSpec((1, _E), lambda b, r: (0, 0)),
        ],
        out_specs=pl.BlockSpec((1, _R, _E), lambda b, r: (b, r, 0)),
        out_shape=jax.ShapeDtypeStruct((_B, _N, _E), jnp.float32),
    )(x, xt, x, W1, wcv, W2, b1p, bcv)
